# baseline (speedup 1.0000x reference)
import functools

import jax
import jax.numpy as jnp
from jax import lax
from jax.experimental import pallas as pl
from jax.experimental.pallas import tpu as pltpu
from jax.experimental.pallas import tpu_sc as plsc

B, S = 4096, 50
E = 128
NW = 32
G = 2
BG = B // G
TOTAL_G = BG * S
PER_W = TOTAL_G // NW
CHUNK = 128
NJ = PER_W // CHUNK
NBUF = 5
NSTEPS = NJ // NBUF


def _make_sc_gather():
    mesh = plsc.VectorSubcoreMesh(core_axis_name="c", subcore_axis_name="s")

    @functools.partial(
        pl.kernel,
        mesh=mesh,
        out_type=jax.ShapeDtypeStruct((TOTAL_G, E), jnp.float32),
        scratch_types=(
            [pltpu.VMEM((NJ, CHUNK), jnp.int32)]
            + [pltpu.VMEM((CHUNK, E), jnp.float32) for _ in range(NBUF)]
            + [pltpu.SemaphoreType.DMA for _ in range(2 * NBUF)]
        ),
    )
    def k(idx_hbm, table_hbm, out_hbm, idx_v, *rest):
        bufs = rest[:NBUF]
        gsem = rest[NBUF:2 * NBUF]
        osem = rest[2 * NBUF:]
        wid = lax.axis_index("s") * 2 + lax.axis_index("c")
        base = wid * PER_W
        pltpu.sync_copy(idx_hbm.at[wid], idx_v)

        def dst(j):
            return out_hbm.at[pl.ds(base + j * CHUNK, CHUNK)]

        for b in range(NBUF):
            pltpu.async_copy(table_hbm.at[idx_v.at[b]], bufs[b], gsem[b])

        def body(i, carry):
            j0 = i * NBUF
            for b in range(NBUF):
                j = j0 + b
                pltpu.make_async_copy(
                    table_hbm.at[idx_v.at[j]], bufs[b], gsem[b]).wait()
                pltpu.async_copy(bufs[b], dst(j), osem[b])
            for b in range(NBUF):
                j = j0 + b
                pltpu.make_async_copy(bufs[b], dst(j), osem[b]).wait()
                pltpu.async_copy(
                    table_hbm.at[idx_v.at[j + NBUF]], bufs[b], gsem[b])
            return carry

        lax.fori_loop(0, NSTEPS - 1, body, 0)

        j0 = (NSTEPS - 1) * NBUF
        for b in range(NBUF):
            j = j0 + b
            pltpu.make_async_copy(
                table_hbm.at[idx_v.at[j]], bufs[b], gsem[b]).wait()
            pltpu.async_copy(bufs[b], dst(j), osem[b])
        for b in range(NBUF):
            j = j0 + b
            pltpu.make_async_copy(bufs[b], dst(j), osem[b]).wait()

    return k


_sc_gather = _make_sc_gather()


@jax.jit
def kernel(x, table):
    idx = x.reshape(G, NW, NJ, CHUNK)
    out = jnp.zeros((B, S, E), jnp.float32)
    for g in range(G):
        inter = _sc_gather(idx[g], table)
        out = lax.dynamic_update_slice(
            out, inter.reshape(BG, S, E), (g * BG, 0, 0))
    return out


# speedup vs baseline: 1.3535x; 1.3535x over previous
import functools

import jax
import jax.numpy as jnp
from jax import lax
from jax.experimental import pallas as pl
from jax.experimental.pallas import tpu as pltpu
from jax.experimental.pallas import tpu_sc as plsc

B, S = 4096, 50
E = 128
NW = 32
G = 2
BG = B // G
BPW = BG // NW
RBUF = 8
NSTEP = BPW // RBUF


def _make_sc_gather():
    mesh = plsc.VectorSubcoreMesh(core_axis_name="c", subcore_axis_name="s")

    @functools.partial(
        pl.kernel,
        mesh=mesh,
        out_type=jax.ShapeDtypeStruct((BG, S, E), jnp.float32),
        scratch_types=(
            [pltpu.VMEM((BPW, S), jnp.int32)]
            + [pltpu.VMEM((S, E), jnp.float32) for _ in range(RBUF)]
            + [pltpu.SemaphoreType.DMA for _ in range(2 * RBUF)]
        ),
    )
    def k(idx_hbm, table_hbm, out_hbm, idx_v, *rest):
        bufs = rest[:RBUF]
        gsem = rest[RBUF:2 * RBUF]
        osem = rest[2 * RBUF:]
        wid = lax.axis_index("s") * 2 + lax.axis_index("c")
        b0 = wid * BPW
        pltpu.sync_copy(idx_hbm.at[wid], idx_v)

        for r in range(RBUF):
            pltpu.async_copy(table_hbm.at[idx_v.at[r]], bufs[r], gsem[r])

        def body(i, carry):
            j0 = i * RBUF
            for r in range(RBUF):
                j = j0 + r
                pltpu.make_async_copy(
                    table_hbm.at[idx_v.at[j]], bufs[r], gsem[r]).wait()
                pltpu.async_copy(bufs[r], out_hbm.at[b0 + j], osem[r])
            for r in range(RBUF):
                j = j0 + r
                pltpu.make_async_copy(
                    bufs[r], out_hbm.at[b0 + j], osem[r]).wait()
                pltpu.async_copy(
                    table_hbm.at[idx_v.at[j + RBUF]], bufs[r], gsem[r])
            return carry

        lax.fori_loop(0, NSTEP - 1, body, 0)

        j0 = (NSTEP - 1) * RBUF
        for r in range(RBUF):
            j = j0 + r
            pltpu.make_async_copy(
                table_hbm.at[idx_v.at[j]], bufs[r], gsem[r]).wait()
            pltpu.async_copy(bufs[r], out_hbm.at[b0 + j], osem[r])
        for r in range(RBUF):
            j = j0 + r
            pltpu.make_async_copy(
                bufs[r], out_hbm.at[b0 + j], osem[r]).wait()

    return k


_sc_gather = _make_sc_gather()


@jax.jit
def kernel(x, table):
    idx = x.reshape(G, NW, BPW, S)
    out = jnp.zeros((B, S, E), jnp.float32)
    for g in range(G):
        inter = _sc_gather(idx[g], table)
        out = lax.dynamic_update_slice(out, inter, (g * BG, 0, 0))
    return out
